# Initial kernel scaffold; baseline (speedup 1.0000x reference)
import jax
import jax.numpy as jnp
from jax.experimental import pallas as pl


def kernel(x, edge_index, edge_attr, batch, W1, b1, W2, b2, Wfc, bfc):
    raise NotImplementedError("write your pallas kernel here")


# speedup vs baseline: 6.6169x; 6.6169x over previous
import dataclasses
import functools

import jax
import jax.numpy as jnp
from jax import lax
from jax.experimental import pallas as pl
from jax.experimental.pallas import tpu as pltpu
from jax.experimental.pallas import tpu_sc as plsc

N = 10000
E = 320000
D = 128

NC = 2
NS = 16
L = 16

N_PAD = 10240
E_PAD = 327680
EPT = E_PAD // (NC * NS)
EBLK = 128
RPT = N_PAD // NS

_vector_mesh = plsc.VectorSubcoreMesh(core_axis_name="c", subcore_axis_name="s")

_sc_params = pltpu.CompilerParams()
if "needs_layout_passes" in pltpu.CompilerParams.__dataclass_fields__:
    _sc_params = dataclasses.replace(_sc_params, needs_layout_passes=False)


@functools.partial(
    pl.kernel,
    mesh=_vector_mesh,
    compiler_params=_sc_params,
    out_type=jax.ShapeDtypeStruct((NC, N_PAD), jnp.float32),
    scratch_types=[
        pltpu.VMEM((EBLK,), jnp.int32),
        pltpu.VMEM((EBLK,), jnp.float32),
        pltpu.VMEM((N_PAD,), jnp.float32),
        pltpu.VMEM((RPT,), jnp.float32),
        pltpu.VMEM((RPT,), jnp.float32),
        pltpu.VMEM_SHARED((NS, N_PAD), jnp.float32),
    ],
)
def _deg_kernel(dst_hbm, w_hbm, deg_hbm, dstv, wv, hist, tmp, accv, stage):
    cid = lax.axis_index("c")
    sid = lax.axis_index("s")

    @pl.loop(0, N_PAD, step=L)
    def _(i):
        hist[pl.ds(i, L)] = jnp.zeros((L,), jnp.float32)

    base = (cid * NS + sid) * EPT

    @pl.loop(0, EPT, step=EBLK)
    def _(e0):
        pltpu.sync_copy(dst_hbm.at[pl.ds(base + e0, EBLK)], dstv)
        pltpu.sync_copy(w_hbm.at[pl.ds(base + e0, EBLK)], wv)

        @pl.loop(0, EBLK, step=L)
        def _(k):
            plsc.addupdate_scatter(hist, [dstv[pl.ds(k, L)]], wv[pl.ds(k, L)])

    pltpu.sync_copy(hist, stage.at[sid])
    plsc.subcore_barrier()

    rbase = sid * RPT

    @pl.loop(0, RPT, step=L)
    def _(i):
        accv[pl.ds(i, L)] = jnp.zeros((L,), jnp.float32)

    for p in range(NS):
        pltpu.sync_copy(stage.at[p, pl.ds(rbase, RPT)], tmp)

        @pl.loop(0, RPT, step=L)
        def _(i):
            plsc.addupdate(accv.at[pl.ds(i, L)], tmp[pl.ds(i, L)])

    pltpu.sync_copy(accv, deg_hbm.at[cid, pl.ds(rbase, RPT)])


@functools.partial(
    pl.kernel,
    mesh=_vector_mesh,
    compiler_params=_sc_params,
    out_type=jax.ShapeDtypeStruct((NC, N_PAD, D), jnp.float32),
    scratch_types=[
        pltpu.VMEM((EBLK,), jnp.int32),
        pltpu.VMEM((EBLK,), jnp.int32),
        pltpu.VMEM((EBLK,), jnp.float32),
        pltpu.VMEM((EBLK, D), jnp.float32),
        pltpu.VMEM_SHARED((N_PAD, D), jnp.float32),
        pltpu.SemaphoreType.DMA,
    ],
)
def _agg_kernel(src_hbm, dst_hbm, w_hbm, g_hbm, s_hbm, srcv, dstv, wv, rows,
                acc, sem):
    cid = lax.axis_index("c")
    sid = lax.axis_index("s")

    @pl.loop(0, EBLK)
    def _(r):
        for jj in range(D // L):
            rows[r, pl.ds(jj * L, L)] = jnp.zeros((L,), jnp.float32)

    rbase = sid * RPT

    @pl.loop(0, RPT, step=EBLK)
    def _(r0):
        pltpu.sync_copy(rows, acc.at[pl.ds(rbase + r0, EBLK)])

    plsc.subcore_barrier()

    base = (cid * NS + sid) * EPT

    @pl.loop(0, EPT, step=EBLK)
    def _(e0):
        pltpu.sync_copy(src_hbm.at[pl.ds(base + e0, EBLK)], srcv)
        pltpu.sync_copy(dst_hbm.at[pl.ds(base + e0, EBLK)], dstv)
        pltpu.sync_copy(w_hbm.at[pl.ds(base + e0, EBLK)], wv)
        pltpu.async_copy(g_hbm.at[srcv], rows, sem).wait()

        @pl.loop(0, EBLK)
        def _(k):
            wvec = plsc.load_gather(wv, [jnp.full((L,), k, jnp.int32)])
            for jj in range(D // L):
                sl = pl.ds(jj * L, L)
                rows[k, sl] = rows[k, sl] * wvec

        pltpu.sync_copy(rows, acc.at[dstv], add=True)

    plsc.subcore_barrier()

    @pl.loop(0, RPT, step=EBLK)
    def _(r0):
        pltpu.sync_copy(acc.at[pl.ds(rbase + r0, EBLK)],
                        s_hbm.at[cid, pl.ds(rbase + r0, EBLK)])


def _tc_pre_body(x_ref, w_ref, degp_ref, g_ref, dinv_ref):
    deg = degp_ref[0] + degp_ref[1] + 1.0
    dinv = lax.rsqrt(deg)
    dinv_ref[...] = dinv
    h = jnp.dot(x_ref[...], w_ref[...], preferred_element_type=jnp.float32)
    g_ref[...] = dinv * h


def _tc_mid_body(s0_ref, s1_ref, g_ref, dinv_ref, b_ref, w_ref, g2_ref):
    dinv = dinv_ref[...]
    z = dinv * (s0_ref[...] + s1_ref[...] + g_ref[...]) + b_ref[...]
    z = jnp.maximum(z, 0.0)
    h = jnp.dot(z, w_ref[...], preferred_element_type=jnp.float32)
    g2_ref[...] = dinv * h


def _tc_post_body(s0_ref, s1_ref, g_ref, dinv_ref, b_ref, w_ref, bfc_ref,
                  out_ref):
    dinv = dinv_ref[...]
    z = dinv * (s0_ref[...] + s1_ref[...] + g_ref[...]) + b_ref[...]
    z = jnp.maximum(z, 0.0)
    out_ref[...] = (jnp.dot(z, w_ref[...], preferred_element_type=jnp.float32)
                    + bfc_ref[...])


_f32 = jnp.float32
_tc_pre = pl.pallas_call(
    _tc_pre_body,
    out_shape=(jax.ShapeDtypeStruct((N_PAD, D), _f32),
               jax.ShapeDtypeStruct((N_PAD, 1), _f32)),
)
_tc_mid = pl.pallas_call(
    _tc_mid_body,
    out_shape=jax.ShapeDtypeStruct((N_PAD, D), _f32),
)
_tc_post = pl.pallas_call(
    _tc_post_body,
    out_shape=jax.ShapeDtypeStruct((N_PAD, D), _f32),
)


def kernel(x, edge_index, edge_attr, batch, W1, b1, W2, b2, Wfc, bfc):
    pad_e = E_PAD - E
    src_p = jnp.concatenate([edge_index[0], jnp.zeros((pad_e,), jnp.int32)])
    dst_p = jnp.concatenate([edge_index[1], jnp.zeros((pad_e,), jnp.int32)])
    w_p = jnp.concatenate([edge_attr, jnp.zeros((pad_e,), jnp.float32)])
    x_p = jnp.concatenate([x, jnp.zeros((N_PAD - N, D), jnp.float32)])

    deg_part = _deg_kernel(dst_p, w_p)
    degp = deg_part.reshape(NC, N_PAD, 1)

    g1, dinv = _tc_pre(x_p, W1, degp)
    s1 = _agg_kernel(src_p, dst_p, w_p, g1)
    g2 = _tc_mid(s1[0], s1[1], g1, dinv, b1.reshape(1, D), W2)
    s2 = _agg_kernel(src_p, dst_p, w_p, g2)
    out = _tc_post(s2[0], s2[1], g2, dinv, b2.reshape(1, D), Wfc,
                   bfc.reshape(1, D))
    return out[:N]


# speedup vs baseline: 10.8898x; 1.6458x over previous
import dataclasses
import functools

import jax
import jax.numpy as jnp
from jax import lax
from jax.experimental import pallas as pl
from jax.experimental.pallas import tpu as pltpu
from jax.experimental.pallas import tpu_sc as plsc

N = 10000
E = 320000
D = 128

NC = 2
NS = 16
L = 16
NW = NC * NS

N_PAD = 10240
E_PAD = 327680
EPT = E_PAD // NW
RPT = N_PAD // NS

EBLK = 64
NBLK = EPT // EBLK
NBUF = 4
CHB = 8
NCHUNK = NBLK // CHB

DBLK = 128
DCH = 8
DROWS = EPT // DBLK

_vector_mesh = plsc.VectorSubcoreMesh(core_axis_name="c", subcore_axis_name="s")

_sc_params = pltpu.CompilerParams()
if "needs_layout_passes" in pltpu.CompilerParams.__dataclass_fields__:
    _sc_params = dataclasses.replace(_sc_params, needs_layout_passes=False)


@functools.partial(
    pl.kernel,
    mesh=_vector_mesh,
    compiler_params=_sc_params,
    out_type=jax.ShapeDtypeStruct((NW, N_PAD), jnp.float32),
    scratch_types=[
        pltpu.VMEM((DCH, DBLK), jnp.int32),
        pltpu.VMEM((DCH, DBLK), jnp.float32),
        pltpu.VMEM((N_PAD,), jnp.float32),
    ],
)
def _deg_kernel(dst_hbm, w_hbm, deg_hbm, dstv, wv, hist):
    cid = lax.axis_index("c")
    sid = lax.axis_index("s")
    wid = cid * NS + sid

    @pl.loop(0, N_PAD, step=L)
    def _(i):
        hist[pl.ds(i, L)] = jnp.zeros((L,), jnp.float32)

    @pl.loop(0, DROWS, step=DCH)
    def _(r0):
        pltpu.sync_copy(dst_hbm.at[pl.ds(wid * DROWS + r0, DCH)], dstv)
        pltpu.sync_copy(w_hbm.at[pl.ds(wid * DROWS + r0, DCH)], wv)

        @pl.loop(0, DCH)
        def _(r):
            @pl.loop(0, DBLK, step=L)
            def _(k):
                plsc.addupdate_scatter(hist, [dstv[r, pl.ds(k, L)]],
                                       wv[r, pl.ds(k, L)])

    pltpu.sync_copy(hist, deg_hbm.at[wid])


@functools.partial(
    pl.kernel,
    mesh=_vector_mesh,
    compiler_params=_sc_params,
    out_type=jax.ShapeDtypeStruct((NC, N_PAD, D), jnp.float32),
    scratch_types=[
        pltpu.VMEM((2, CHB, EBLK), jnp.int32),
        pltpu.VMEM((2, CHB, EBLK), jnp.int32),
        pltpu.VMEM((2, CHB, EBLK), jnp.float32),
        pltpu.VMEM((NBUF, EBLK, D), jnp.float32),
        pltpu.VMEM_SHARED((N_PAD, D), jnp.float32),
        pltpu.SemaphoreType.DMA,
        pltpu.SemaphoreType.DMA,
        pltpu.SemaphoreType.DMA,
        pltpu.SemaphoreType.DMA,
        pltpu.SemaphoreType.DMA,
        pltpu.SemaphoreType.DMA,
        pltpu.SemaphoreType.DMA,
        pltpu.SemaphoreType.DMA,
        pltpu.SemaphoreType.DMA,
        pltpu.SemaphoreType.DMA,
    ],
)
def _agg_kernel(src_hbm, dst_hbm, w_hbm, g_hbm, s_hbm, srcv, dstv, wv, rows,
                acc, g0, g1, g2, g3, s0, s1, s2, s3, c0, c1):
    cid = lax.axis_index("c")
    sid = lax.axis_index("s")
    wid = cid * NS + sid
    gsem = [g0, g1, g2, g3]
    ssem = [s0, s1, s2, s3]
    csem = [c0, c1]
    ebase = wid * NBLK

    def chunk_fire(c, cc):
        sl = pl.ds(ebase + c * CHB, CHB)
        pltpu.async_copy(src_hbm.at[sl], srcv.at[cc], csem[cc])
        pltpu.async_copy(dst_hbm.at[sl], dstv.at[cc], csem[cc])
        pltpu.async_copy(w_hbm.at[sl], wv.at[cc], csem[cc])

    def chunk_drain(cc):
        sl = pl.ds(ebase, CHB)
        pltpu.make_async_copy(src_hbm.at[sl], srcv.at[cc], csem[cc]).wait()
        pltpu.make_async_copy(dst_hbm.at[sl], dstv.at[cc], csem[cc]).wait()
        pltpu.make_async_copy(w_hbm.at[sl], wv.at[cc], csem[cc]).wait()

    @pl.loop(0, EBLK)
    def _(r):
        for jj in range(D // L):
            rows[0, r, pl.ds(jj * L, L)] = jnp.zeros((L,), jnp.float32)

    rbase = sid * RPT

    @pl.loop(0, RPT, step=EBLK)
    def _(r0):
        pltpu.sync_copy(rows.at[0], acc.at[pl.ds(rbase + r0, EBLK)])

    plsc.subcore_barrier()

    chunk_fire(0, 0)
    chunk_drain(0)
    pltpu.async_copy(g_hbm.at[srcv.at[0, 0]], rows.at[0], gsem[0])
    pltpu.async_copy(g_hbm.at[srcv.at[0, 1]], rows.at[1], gsem[1])

    @pl.loop(0, NCHUNK, step=2)
    def _(ch0):
        for cc in range(2):
            c = ch0 + cc
            ncc = (cc + 1) % 2
            for j in range(CHB):
                b = c * CHB + j
                rj = j % NBUF
                jp = (rj + 2) % NBUF

                if j == 1:
                    @pl.when(c + 1 < NCHUNK)
                    def _():
                        chunk_fire(c + 1, ncc)

                if j == CHB - 2:
                    @pl.when(c + 1 < NCHUNK)
                    def _():
                        chunk_drain(ncc)

                @pl.when(b >= 2)
                def _():
                    pltpu.make_async_copy(rows.at[jp], acc.at[dstv.at[0, 0]],
                                          ssem[jp]).wait()

                @pl.when(b + 2 < NBLK)
                def _():
                    if j < CHB - 2:
                        idx = srcv.at[cc, j + 2]
                    else:
                        idx = srcv.at[ncc, j - (CHB - 2)]
                    pltpu.async_copy(g_hbm.at[idx], rows.at[jp], gsem[jp])

                pltpu.make_async_copy(g_hbm.at[srcv.at[cc, j]], rows.at[rj],
                                      gsem[rj]).wait()

                @pl.loop(0, EBLK)
                def _(k):
                    wvec = plsc.load_gather(
                        wv, [jnp.full((L,), cc, jnp.int32),
                             jnp.full((L,), j, jnp.int32),
                             jnp.full((L,), k, jnp.int32)])
                    for jj in range(D // L):
                        sl = pl.ds(jj * L, L)
                        rows[rj, k, sl] = rows[rj, k, sl] * wvec

                pltpu.async_copy(rows.at[rj], acc.at[dstv.at[cc, j]],
                                 ssem[rj], add=True)

    pltpu.make_async_copy(rows.at[2], acc.at[dstv.at[0, 0]], ssem[2]).wait()
    pltpu.make_async_copy(rows.at[3], acc.at[dstv.at[0, 0]], ssem[3]).wait()
    plsc.subcore_barrier()

    pltpu.sync_copy(acc.at[pl.ds(rbase, RPT)],
                    s_hbm.at[cid, pl.ds(rbase, RPT)])


def _tc_pre_body(x_ref, w_ref, degt_ref, g_ref, dinv_ref):
    deg = jnp.sum(degt_ref[...], axis=1, keepdims=True) + 1.0
    dinv = lax.rsqrt(deg)
    dinv_ref[...] = dinv
    h = jnp.dot(x_ref[...], w_ref[...], preferred_element_type=jnp.float32)
    g_ref[...] = dinv * h


def _tc_mid_body(s0_ref, s1_ref, g_ref, dinv_ref, b_ref, w_ref, g2_ref):
    dinv = dinv_ref[...]
    z = dinv * (s0_ref[...] + s1_ref[...] + g_ref[...]) + b_ref[...]
    z = jnp.maximum(z, 0.0)
    h = jnp.dot(z, w_ref[...], preferred_element_type=jnp.float32)
    g2_ref[...] = dinv * h


def _tc_post_body(s0_ref, s1_ref, g_ref, dinv_ref, b_ref, w_ref, bfc_ref,
                  out_ref):
    dinv = dinv_ref[...]
    z = dinv * (s0_ref[...] + s1_ref[...] + g_ref[...]) + b_ref[...]
    z = jnp.maximum(z, 0.0)
    out_ref[...] = (jnp.dot(z, w_ref[...], preferred_element_type=jnp.float32)
                    + bfc_ref[...])


_f32 = jnp.float32
_tc_pre = pl.pallas_call(
    _tc_pre_body,
    out_shape=(jax.ShapeDtypeStruct((N_PAD, D), _f32),
               jax.ShapeDtypeStruct((N_PAD, 1), _f32)),
)
_tc_mid = pl.pallas_call(
    _tc_mid_body,
    out_shape=jax.ShapeDtypeStruct((N_PAD, D), _f32),
)
_tc_post = pl.pallas_call(
    _tc_post_body,
    out_shape=jax.ShapeDtypeStruct((N_PAD, D), _f32),
)


def kernel(x, edge_index, edge_attr, batch, W1, b1, W2, b2, Wfc, bfc):
    pad_e = E_PAD - E
    src_p = jnp.concatenate([edge_index[0], jnp.zeros((pad_e,), jnp.int32)])
    dst_p = jnp.concatenate([edge_index[1], jnp.zeros((pad_e,), jnp.int32)])
    w_p = jnp.concatenate([edge_attr, jnp.zeros((pad_e,), jnp.float32)])
    src64 = src_p.reshape(E_PAD // EBLK, EBLK)
    dst64 = dst_p.reshape(E_PAD // EBLK, EBLK)
    w64 = w_p.reshape(E_PAD // EBLK, EBLK)
    dst128 = dst_p.reshape(E_PAD // DBLK, DBLK)
    w128 = w_p.reshape(E_PAD // DBLK, DBLK)
    x_p = jnp.concatenate([x, jnp.zeros((N_PAD - N, D), jnp.float32)])

    deg_part = _deg_kernel(dst128, w128)
    degt = deg_part.T

    g1, dinv = _tc_pre(x_p, W1, degt)
    s1 = _agg_kernel(src64, dst64, w64, g1)
    g2 = _tc_mid(s1[0], s1[1], g1, dinv, b1.reshape(1, D), W2)
    s2 = _agg_kernel(src64, dst64, w64, g2)
    out = _tc_post(s2[0], s2[1], g2, dinv, b2.reshape(1, D), Wfc,
                   bfc.reshape(1, D))
    return out[:N]
